# baseline (speedup 1.0000x reference)
import jax
import jax.numpy as jnp
from jax.experimental import pallas as pl
from jax.experimental.pallas import tpu as pltpu

_SLOTS = 16
_WB = 8
_NSPLIT = 2


def _body(x_ref, act_ref, bat_ref, nl_ref,
          w1_ref, b1_ref, w2_ref, b2_ref,
          wr1_ref, br1_ref, wr2_ref, br2_ref,
          out_ref, acc_ref, cnt_ref):
    i = pl.program_id(0)
    R = x_ref.shape[0]
    B = out_ref.shape[0]

    @pl.when(i == 0)
    def _init():
        acc_ref[...] = jnp.zeros_like(acc_ref)
        cnt_ref[...] = jnp.zeros_like(cnt_ref)

    H = R // _NSPLIT
    for h in range(_NSPLIT):
        s0 = h * H
        a = jnp.maximum(
            jnp.dot(x_ref[pl.ds(s0, H), :].astype(jnp.bfloat16), w1_ref[...],
                    preferred_element_type=jnp.float32)
            + b1_ref[...], 0.0)
        a16 = a.astype(jnp.bfloat16)

        act = act_ref[0, :, pl.ds(s0, H)]
        bat = bat_ref[0, :, pl.ds(s0, H)]
        bmin = bat_ref[0, 0, s0]
        bmax = bat_ref[0, 0, s0 + H - 1]

        @pl.when(bmin == bmax)
        def _single_batch(a16=a16, act=act, bmin=bmin):
            act_iota = jax.lax.broadcasted_iota(jnp.int32, (_SLOTS, H), 0)
            onehot = jnp.where(act == act_iota, 1.0, 0.0)
            partial = jnp.dot(onehot.astype(jnp.bfloat16), a16,
                              preferred_element_type=jnp.float32)
            c = jnp.sum(onehot, axis=1, keepdims=True)
            off = bmin * _SLOTS
            acc_ref[pl.ds(off, _SLOTS), :] += partial
            cnt_ref[pl.ds(off, _SLOTS), :] += jnp.broadcast_to(c, (_SLOTS, 128))

        @pl.when(bmin != bmax)
        def _multi_batch(a16=a16, act=act, bat=bat, bmin=bmin, bmax=bmax):
            W_ROWS = _WB * _SLOTS
            row_iota = jax.lax.broadcasted_iota(jnp.int32, (W_ROWS, H), 0)
            nwin = (bmax - bmin) // _WB + 1

            def win_body(w, carry):
                base = bmin + w * _WB
                key = (bat - base) * _SLOTS + act
                onehot = jnp.where(key == row_iota, 1.0, 0.0)
                partial = jnp.dot(onehot.astype(jnp.bfloat16), a16,
                                  preferred_element_type=jnp.float32)
                c = jnp.sum(onehot, axis=1, keepdims=True)
                off = base * _SLOTS
                acc_ref[pl.ds(off, W_ROWS), :] += partial
                cnt_ref[pl.ds(off, W_ROWS), :] += jnp.broadcast_to(c, (W_ROWS, 128))
                return carry

            jax.lax.fori_loop(0, nwin, win_body, 0)

    @pl.when(i == pl.num_programs(0) - 1)
    def _finish():
        nl = nl_ref[0, 0]
        S = B * _SLOTS
        xsum = (jnp.dot(acc_ref[pl.ds(0, S), :], w2_ref[...],
                        preferred_element_type=jnp.float32)
                + cnt_ref[pl.ds(0, S), :] * b2_ref[...])
        r = jnp.maximum(
            jnp.dot(xsum, wr1_ref[...], preferred_element_type=jnp.float32)
            + br1_ref[...], 0.0)
        r = (jnp.dot(r, wr2_ref[...], preferred_element_type=jnp.float32)
             + br2_ref[...])
        r3 = r.reshape(B, _SLOTS, r.shape[-1])
        lidx = jax.lax.broadcasted_iota(jnp.int32, r3.shape, 1)
        out_ref[...] = jnp.sum(jnp.where(lidx < nl, r3, 0.0), axis=1)


def kernel(x, activation_idx, batch_idx, num_layers,
           W_phi1, b_phi1, W_phi2, b_phi2,
           W_rho1, b_rho1, W_rho2, b_rho2):
    N, D_IN = x.shape
    D_OUT = W_rho2.shape[1]
    B = 64

    R = 32000
    while N % R:
        R //= 2
    G = N // R

    act3 = activation_idx.astype(jnp.int32).reshape(G, 1, R)
    bat3 = batch_idx.astype(jnp.int32).reshape(G, 1, R)
    nl = jnp.asarray(num_layers, jnp.int32).reshape(1, 1)

    full = lambda shp: pl.BlockSpec(shp, lambda i: (0,) * len(shp))
    out = pl.pallas_call(
        _body,
        grid=(G,),
        in_specs=[
            pl.BlockSpec((R, D_IN), lambda i: (i, 0)),
            pl.BlockSpec((1, 1, R), lambda i: (i, 0, 0)),
            pl.BlockSpec((1, 1, R), lambda i: (i, 0, 0)),
            pl.BlockSpec(memory_space=pltpu.SMEM),
            full((D_IN, D_IN)), full((1, D_IN)),
            full((D_IN, D_IN)), full((1, D_IN)),
            full((D_IN, D_IN)), full((1, D_IN)),
            full((D_IN, D_OUT)), full((1, D_OUT)),
        ],
        out_specs=pl.BlockSpec((B, D_OUT), lambda i: (0, 0)),
        out_shape=jax.ShapeDtypeStruct((B, D_OUT), jnp.float32),
        scratch_shapes=[
            pltpu.VMEM(((B + _WB) * _SLOTS, D_IN), jnp.float32),
            pltpu.VMEM(((B + _WB) * _SLOTS, D_IN), jnp.float32),
        ],
        compiler_params=pltpu.CompilerParams(
            dimension_semantics=("arbitrary",)),
    )(x, act3, bat3, nl,
      W_phi1.astype(jnp.bfloat16), b_phi1.reshape(1, -1),
      W_phi2, b_phi2.reshape(1, -1),
      W_rho1, b_rho1.reshape(1, -1), W_rho2, b_rho2.reshape(1, -1))
    return out


# speedup vs baseline: 1.0245x; 1.0245x over previous
import jax
import jax.numpy as jnp
from jax.experimental import pallas as pl
from jax.experimental.pallas import tpu as pltpu

_SLOTS = 16
_WB = 4
_NSPLIT = 1


def _body(x_ref, act_ref, bat_ref, nl_ref,
          w1_ref, b1_ref, w2_ref, b2_ref,
          wr1_ref, br1_ref, wr2_ref, br2_ref,
          out_ref, acc_ref, cnt_ref):
    i = pl.program_id(0)
    R = x_ref.shape[0]
    B = out_ref.shape[0]

    @pl.when(i == 0)
    def _init():
        acc_ref[...] = jnp.zeros_like(acc_ref)
        cnt_ref[...] = jnp.zeros_like(cnt_ref)

    H = R // _NSPLIT
    for h in range(_NSPLIT):
        s0 = h * H
        a = jnp.maximum(
            jnp.dot(x_ref[pl.ds(s0, H), :].astype(jnp.bfloat16), w1_ref[...],
                    preferred_element_type=jnp.float32)
            + b1_ref[...], 0.0)
        a16 = a.astype(jnp.bfloat16)

        act = act_ref[0, :, pl.ds(s0, H)]
        bat = bat_ref[0, :, pl.ds(s0, H)]
        bmin = bat_ref[0, 0, s0]
        bmax = bat_ref[0, 0, s0 + H - 1]

        @pl.when(bmin == bmax)
        def _single_batch(a16=a16, act=act, bmin=bmin):
            act_iota = jax.lax.broadcasted_iota(jnp.int32, (_SLOTS, H), 0)
            onehot = jnp.where(act == act_iota, 1.0, 0.0)
            partial = jnp.dot(onehot.astype(jnp.bfloat16), a16,
                              preferred_element_type=jnp.float32)
            c = jnp.sum(onehot, axis=1, keepdims=True)
            off = bmin * _SLOTS
            acc_ref[pl.ds(off, _SLOTS), :] += partial
            cnt_ref[pl.ds(off, _SLOTS), :] += jnp.broadcast_to(c, (_SLOTS, 128))

        @pl.when(bmin != bmax)
        def _multi_batch(a16=a16, act=act, bat=bat, bmin=bmin, bmax=bmax):
            W_ROWS = _WB * _SLOTS
            row_iota = jax.lax.broadcasted_iota(jnp.int32, (W_ROWS, H), 0)
            nwin = (bmax - bmin) // _WB + 1

            def win_body(w, carry):
                base = bmin + w * _WB
                key = (bat - base) * _SLOTS + act
                onehot = jnp.where(key == row_iota, 1.0, 0.0)
                partial = jnp.dot(onehot.astype(jnp.bfloat16), a16,
                                  preferred_element_type=jnp.float32)
                c = jnp.sum(onehot, axis=1, keepdims=True)
                off = base * _SLOTS
                acc_ref[pl.ds(off, W_ROWS), :] += partial
                cnt_ref[pl.ds(off, W_ROWS), :] += jnp.broadcast_to(c, (W_ROWS, 128))
                return carry

            jax.lax.fori_loop(0, nwin, win_body, 0)

    @pl.when(i == pl.num_programs(0) - 1)
    def _finish():
        nl = nl_ref[0, 0]
        S = B * _SLOTS
        xsum = (jnp.dot(acc_ref[pl.ds(0, S), :], w2_ref[...],
                        preferred_element_type=jnp.float32)
                + cnt_ref[pl.ds(0, S), :] * b2_ref[...])
        r = jnp.maximum(
            jnp.dot(xsum, wr1_ref[...], preferred_element_type=jnp.float32)
            + br1_ref[...], 0.0)
        r = (jnp.dot(r, wr2_ref[...], preferred_element_type=jnp.float32)
             + br2_ref[...])
        r3 = r.reshape(B, _SLOTS, r.shape[-1])
        lidx = jax.lax.broadcasted_iota(jnp.int32, r3.shape, 1)
        out_ref[...] = jnp.sum(jnp.where(lidx < nl, r3, 0.0), axis=1)


def kernel(x, activation_idx, batch_idx, num_layers,
           W_phi1, b_phi1, W_phi2, b_phi2,
           W_rho1, b_rho1, W_rho2, b_rho2):
    N, D_IN = x.shape
    D_OUT = W_rho2.shape[1]
    B = 64

    R = 32000
    while N % R:
        R //= 2
    G = N // R

    act3 = activation_idx.astype(jnp.int32).reshape(G, 1, R)
    bat3 = batch_idx.astype(jnp.int32).reshape(G, 1, R)
    nl = jnp.asarray(num_layers, jnp.int32).reshape(1, 1)

    full = lambda shp: pl.BlockSpec(shp, lambda i: (0,) * len(shp))
    out = pl.pallas_call(
        _body,
        grid=(G,),
        in_specs=[
            pl.BlockSpec((R, D_IN), lambda i: (i, 0)),
            pl.BlockSpec((1, 1, R), lambda i: (i, 0, 0)),
            pl.BlockSpec((1, 1, R), lambda i: (i, 0, 0)),
            pl.BlockSpec(memory_space=pltpu.SMEM),
            full((D_IN, D_IN)), full((1, D_IN)),
            full((D_IN, D_IN)), full((1, D_IN)),
            full((D_IN, D_IN)), full((1, D_IN)),
            full((D_IN, D_OUT)), full((1, D_OUT)),
        ],
        out_specs=pl.BlockSpec((B, D_OUT), lambda i: (0, 0)),
        out_shape=jax.ShapeDtypeStruct((B, D_OUT), jnp.float32),
        scratch_shapes=[
            pltpu.VMEM(((B + _WB) * _SLOTS, D_IN), jnp.float32),
            pltpu.VMEM(((B + _WB) * _SLOTS, D_IN), jnp.float32),
        ],
        compiler_params=pltpu.CompilerParams(
            dimension_semantics=("arbitrary",)),
    )(x, act3, bat3, nl,
      W_phi1.astype(jnp.bfloat16), b_phi1.reshape(1, -1),
      W_phi2, b_phi2.reshape(1, -1),
      W_rho1, b_rho1.reshape(1, -1), W_rho2, b_rho2.reshape(1, -1))
    return out


# speedup vs baseline: 1.0617x; 1.0363x over previous
import jax
import jax.numpy as jnp
from jax.experimental import pallas as pl
from jax.experimental.pallas import tpu as pltpu

_SLOTS = 16
_WB = 4
_NSPLIT = 2


def _body(x_ref, act_ref, bat_ref, nl_ref,
          w1_ref, b1_ref, w2_ref, b2_ref,
          wr1_ref, br1_ref, wr2_ref, br2_ref,
          out_ref, acc_ref, cnt_ref):
    i = pl.program_id(0)
    R = x_ref.shape[0]
    B = out_ref.shape[0]

    @pl.when(i == 0)
    def _init():
        acc_ref[...] = jnp.zeros_like(acc_ref)
        cnt_ref[...] = jnp.zeros_like(cnt_ref)

    H = R // _NSPLIT
    for h in range(_NSPLIT):
        s0 = h * H
        a = jnp.maximum(
            jnp.dot(x_ref[pl.ds(s0, H), :].astype(jnp.bfloat16), w1_ref[...],
                    preferred_element_type=jnp.float32)
            + b1_ref[...], 0.0)
        a16 = a.astype(jnp.bfloat16)

        act = act_ref[0, :, pl.ds(s0, H)]
        bat = bat_ref[0, :, pl.ds(s0, H)]
        bmin = bat_ref[0, 0, s0]
        bmax = bat_ref[0, 0, s0 + H - 1]

        @pl.when(bmin == bmax)
        def _single_batch(a16=a16, act=act, bmin=bmin):
            act_iota = jax.lax.broadcasted_iota(jnp.int32, (_SLOTS, H), 0)
            onehot = jnp.where(act == act_iota, 1.0, 0.0)
            partial = jnp.dot(onehot.astype(jnp.bfloat16), a16,
                              preferred_element_type=jnp.float32)
            c = jnp.sum(onehot, axis=1, keepdims=True)
            off = bmin * _SLOTS
            acc_ref[pl.ds(off, _SLOTS), :] += partial
            cnt_ref[pl.ds(off, _SLOTS), :] += jnp.broadcast_to(c, (_SLOTS, 128))

        @pl.when(bmin != bmax)
        def _multi_batch(a16=a16, act=act, bat=bat, bmin=bmin, bmax=bmax):
            W_ROWS = _WB * _SLOTS
            row_iota = jax.lax.broadcasted_iota(jnp.int32, (W_ROWS, H), 0)
            nwin = (bmax - bmin) // _WB + 1

            def win_body(w, carry):
                base = bmin + w * _WB
                key = (bat - base) * _SLOTS + act
                onehot = jnp.where(key == row_iota, 1.0, 0.0)
                partial = jnp.dot(onehot.astype(jnp.bfloat16), a16,
                                  preferred_element_type=jnp.float32)
                c = jnp.sum(onehot, axis=1, keepdims=True)
                off = base * _SLOTS
                acc_ref[pl.ds(off, W_ROWS), :] += partial
                cnt_ref[pl.ds(off, W_ROWS), :] += jnp.broadcast_to(c, (W_ROWS, 128))
                return carry

            jax.lax.fori_loop(0, nwin, win_body, 0)

    @pl.when(i == pl.num_programs(0) - 1)
    def _finish():
        nl = nl_ref[0, 0]
        S = B * _SLOTS
        xsum = (jnp.dot(acc_ref[pl.ds(0, S), :], w2_ref[...],
                        preferred_element_type=jnp.float32)
                + cnt_ref[pl.ds(0, S), :] * b2_ref[...])
        r = jnp.maximum(
            jnp.dot(xsum, wr1_ref[...], preferred_element_type=jnp.float32)
            + br1_ref[...], 0.0)
        r = (jnp.dot(r, wr2_ref[...], preferred_element_type=jnp.float32)
             + br2_ref[...])
        r3 = r.reshape(B, _SLOTS, r.shape[-1])
        lidx = jax.lax.broadcasted_iota(jnp.int32, r3.shape, 1)
        out_ref[...] = jnp.sum(jnp.where(lidx < nl, r3, 0.0), axis=1)


def kernel(x, activation_idx, batch_idx, num_layers,
           W_phi1, b_phi1, W_phi2, b_phi2,
           W_rho1, b_rho1, W_rho2, b_rho2):
    N, D_IN = x.shape
    D_OUT = W_rho2.shape[1]
    B = 64

    R = 32000
    while N % R:
        R //= 2
    G = N // R

    act3 = activation_idx.astype(jnp.int32).reshape(G, 1, R)
    bat3 = batch_idx.astype(jnp.int32).reshape(G, 1, R)
    nl = jnp.asarray(num_layers, jnp.int32).reshape(1, 1)

    full = lambda shp: pl.BlockSpec(shp, lambda i: (0,) * len(shp))
    out = pl.pallas_call(
        _body,
        grid=(G,),
        in_specs=[
            pl.BlockSpec((R, D_IN), lambda i: (i, 0)),
            pl.BlockSpec((1, 1, R), lambda i: (i, 0, 0)),
            pl.BlockSpec((1, 1, R), lambda i: (i, 0, 0)),
            pl.BlockSpec(memory_space=pltpu.SMEM),
            full((D_IN, D_IN)), full((1, D_IN)),
            full((D_IN, D_IN)), full((1, D_IN)),
            full((D_IN, D_IN)), full((1, D_IN)),
            full((D_IN, D_OUT)), full((1, D_OUT)),
        ],
        out_specs=pl.BlockSpec((B, D_OUT), lambda i: (0, 0)),
        out_shape=jax.ShapeDtypeStruct((B, D_OUT), jnp.float32),
        scratch_shapes=[
            pltpu.VMEM(((B + _WB) * _SLOTS, D_IN), jnp.float32),
            pltpu.VMEM(((B + _WB) * _SLOTS, D_IN), jnp.float32),
        ],
        compiler_params=pltpu.CompilerParams(
            dimension_semantics=("arbitrary",)),
    )(x, act3, bat3, nl,
      W_phi1.astype(jnp.bfloat16), b_phi1.reshape(1, -1),
      W_phi2, b_phi2.reshape(1, -1),
      W_rho1, b_rho1.reshape(1, -1), W_rho2, b_rho2.reshape(1, -1))
    return out
